# Initial kernel scaffold; baseline (speedup 1.0000x reference)
import jax
import jax.numpy as jnp
from jax.experimental import pallas as pl


def kernel(mobility_x, mobility_edge_index, edar_x, edar_edge_index, edar_muni_mask, params):
    raise NotImplementedError("write your pallas kernel here")


# speedup vs baseline: 4.8765x; 4.8765x over previous
import functools

import jax
import jax.numpy as jnp
from jax import lax
from jax.experimental import pallas as pl
from jax.experimental.pallas import tpu as pltpu
from jax.experimental.pallas import tpu_sc as plsc

NC = 2
NS = 16
L = 16
NW = NC * NS
CH = 128


@functools.lru_cache(maxsize=None)
def _seg_sum_sc(n_nodes, dim, n_edges):
    assert n_nodes % 8 == 0 and n_edges % CH == 0 and dim % L == 0
    full = (-(-n_nodes // NS) + 7) // 8 * 8
    tail = n_nodes - full * (NS - 1)
    assert 0 < tail <= full
    n_chunks = n_edges // CH
    base_chunks = n_chunks // NW
    rem = n_chunks % NW

    mesh = plsc.VectorSubcoreMesh(core_axis_name="c", subcore_axis_name="s",
                                  num_cores=NC, num_subcores=NS)

    @functools.partial(
        pl.kernel,
        mesh=mesh,
        compiler_params=pltpu.CompilerParams(needs_layout_passes=False),
        out_type=[
            jax.ShapeDtypeStruct((NC, n_nodes, dim), jnp.float32),
            jax.ShapeDtypeStruct((NW * n_nodes,), jnp.float32),
        ],
        scratch_types=[
            pltpu.VMEM((CH,), jnp.int32),
            pltpu.VMEM((CH,), jnp.int32),
            pltpu.VMEM((CH, dim), jnp.float32),
            pltpu.VMEM((n_nodes,), jnp.float32),
            pltpu.VMEM_SHARED((n_nodes, dim), jnp.float32),
            pltpu.SemaphoreType.DMA,
        ],
    )
    def k(x_hbm, src_hbm, dst_hbm, zeros_hbm, agg_out, deg_out,
          srcb, dstb, rows, degv, agg_sh, sem):
        c = lax.axis_index("c")
        s = lax.axis_index("s")
        wid = s * NC + c

        zero16 = jnp.zeros((L,), jnp.float32)

        def zdeg(i, _):
            degv[pl.ds(i * L, L)] = zero16
            return 0

        lax.fori_loop(0, n_nodes // L, zdeg, 0)

        r0 = s * full

        @pl.when(s < NS - 1)
        def _():
            pltpu.sync_copy(zeros_hbm.at[pl.ds(r0, full)],
                            agg_sh.at[pl.ds(r0, full)])

        @pl.when(s == NS - 1)
        def _():
            pltpu.sync_copy(zeros_hbm.at[pl.ds(r0, tail)],
                            agg_sh.at[pl.ds(r0, tail)])

        plsc.subcore_barrier()

        my_n = base_chunks + jnp.where(wid < rem, 1, 0)
        my_first = wid * base_chunks + jnp.minimum(wid, rem)
        ones16 = jnp.ones((L,), jnp.float32)

        def body(i, _):
            ebase = (my_first + i) * CH
            pltpu.sync_copy(src_hbm.at[pl.ds(ebase, CH)], srcb)
            pltpu.sync_copy(dst_hbm.at[pl.ds(ebase, CH)], dstb)
            pltpu.async_copy(x_hbm.at[srcb], rows, sem).wait()
            pltpu.sync_copy(rows, agg_sh.at[dstb], add=True)
            for j in range(CH // L):
                idx = dstb[pl.ds(j * L, L)]
                plsc.addupdate_scatter(degv, [idx], ones16)
            return 0

        lax.fori_loop(0, my_n, body, 0)

        pltpu.sync_copy(degv, deg_out.at[pl.ds(wid * n_nodes, n_nodes)])
        plsc.subcore_barrier()

        @pl.when(s < NS - 1)
        def _():
            pltpu.sync_copy(agg_sh.at[pl.ds(r0, full)],
                            agg_out.at[c, pl.ds(r0, full)])

        @pl.when(s == NS - 1)
        def _():
            pltpu.sync_copy(agg_sh.at[pl.ds(r0, tail)],
                            agg_out.at[c, pl.ds(r0, tail)])

    return k


def _sage_dense(x, aggp, degp, ws, wn, b, residual):
    n, din = x.shape
    dout = ws.shape[1]
    r = 512
    grid = (pl.cdiv(n, r),)

    def body(x_ref, a_ref, d_ref, ws_ref, wn_ref, b_ref, o_ref):
        agg = a_ref[0] + a_ref[1]
        deg = jnp.sum(d_ref[...], axis=0)[:, None]
        mean = agg / jnp.maximum(deg, 1.0)
        h = jnp.dot(x_ref[...], ws_ref[...], preferred_element_type=jnp.float32)
        h = h + jnp.dot(mean, wn_ref[...], preferred_element_type=jnp.float32)
        h = jnp.maximum(h + b_ref[...], 0.0)
        if residual:
            h = h + x_ref[...]
        o_ref[...] = h

    return pl.pallas_call(
        body,
        grid=grid,
        in_specs=[
            pl.BlockSpec((r, din), lambda i: (i, 0)),
            pl.BlockSpec((NC, r, din), lambda i: (0, i, 0)),
            pl.BlockSpec((NW, r), lambda i: (0, i)),
            pl.BlockSpec((din, dout), lambda i: (0, 0)),
            pl.BlockSpec((din, dout), lambda i: (0, 0)),
            pl.BlockSpec((1, dout), lambda i: (0, 0)),
        ],
        out_specs=pl.BlockSpec((r, dout), lambda i: (i, 0)),
        out_shape=jax.ShapeDtypeStruct((n, dout), jnp.float32),
    )(x, aggp, degp, ws, wn, b.reshape(1, dout))


def _attn_fuse(mob, ed, mask, p):
    n_mob, o_mob = mob.shape
    n_ed, o_ed = ed.shape
    r = 512
    grid = (pl.cdiv(n_mob, r),)

    def body(mob_ref, ed_ref, mask_ref, wq_ref, wk_ref, alpha_ref,
             wf1_ref, bf1_ref, wf2_ref, bf2_ref, wg_ref, bg_ref,
             lng_ref, lnb_ref, o_ref):
        m = mob_ref[...]
        ed_f = ed_ref[...]
        q = jnp.dot(m, wq_ref[...], preferred_element_type=jnp.float32)
        kk = jnp.dot(ed_f, wk_ref[...], preferred_element_type=jnp.float32)
        scores = lax.dot_general(
            q, kk, (((1,), (1,)), ((), ())),
            preferred_element_type=jnp.float32)
        smax = jnp.max(scores, axis=1, keepdims=True)
        e = jnp.exp(scores - smax)
        learned = e / jnp.sum(e, axis=1, keepdims=True)
        msk = mask_ref[...]
        mask_norm = msk / jnp.maximum(jnp.sum(msk, axis=1, keepdims=True), 1e-8)
        g = 1.0 / (1.0 + jnp.exp(-alpha_ref[0, 0]))
        attn = g * mask_norm + (1.0 - g) * learned
        sig = jnp.dot(attn, ed_f, preferred_element_type=jnp.float32)

        wf1 = wf1_ref[...]
        h1 = (jnp.dot(m, wf1[:o_mob, :], preferred_element_type=jnp.float32)
              + jnp.dot(sig, wf1[o_mob:o_mob + o_ed, :],
                        preferred_element_type=jnp.float32) + bf1_ref[...])
        h1 = jnp.maximum(h1, 0.0)
        fused = jnp.dot(h1, wf2_ref[...],
                        preferred_element_type=jnp.float32) + bf2_ref[...]
        wg = wg_ref[...]
        gz = (jnp.dot(m, wg[:o_mob, :], preferred_element_type=jnp.float32)
              + jnp.dot(sig, wg[o_mob:o_mob + o_ed, :],
                        preferred_element_type=jnp.float32) + bg_ref[...])
        gate = 1.0 / (1.0 + jnp.exp(-gz))
        comb = gate * fused + (1.0 - gate) * m
        mu = jnp.mean(comb, axis=-1, keepdims=True)
        var = jnp.mean((comb - mu) ** 2, axis=-1, keepdims=True)
        comb = (comb - mu) / jnp.sqrt(var + 1e-5) * lng_ref[...] + lnb_ref[...]
        nrm = jnp.sqrt(jnp.sum(comb * comb, axis=-1, keepdims=True))
        o_ref[...] = comb / jnp.maximum(nrm, 1e-12)

    o_ed_dim = p['Wk'].shape[1]
    return pl.pallas_call(
        body,
        grid=grid,
        in_specs=[
            pl.BlockSpec((r, o_mob), lambda i: (i, 0)),
            pl.BlockSpec((n_ed, o_ed), lambda i: (0, 0)),
            pl.BlockSpec((r, n_ed), lambda i: (i, 0)),
            pl.BlockSpec(p['Wq'].shape, lambda i: (0, 0)),
            pl.BlockSpec(p['Wk'].shape, lambda i: (0, 0)),
            pl.BlockSpec((1, 1), lambda i: (0, 0)),
            pl.BlockSpec(p['Wf1'].shape, lambda i: (0, 0)),
            pl.BlockSpec((1, o_mob), lambda i: (0, 0)),
            pl.BlockSpec(p['Wf2'].shape, lambda i: (0, 0)),
            pl.BlockSpec((1, o_mob), lambda i: (0, 0)),
            pl.BlockSpec(p['Wg'].shape, lambda i: (0, 0)),
            pl.BlockSpec((1, o_mob), lambda i: (0, 0)),
            pl.BlockSpec((1, o_mob), lambda i: (0, 0)),
            pl.BlockSpec((1, o_mob), lambda i: (0, 0)),
        ],
        out_specs=pl.BlockSpec((r, o_mob), lambda i: (i, 0)),
        out_shape=jax.ShapeDtypeStruct((n_mob, o_mob), jnp.float32),
    )(mob, ed, mask,
      p['Wq'], p['Wk'], p['alpha'].reshape(1, 1),
      p['Wf1'], p['bf1'].reshape(1, o_mob), p['Wf2'], p['bf2'].reshape(1, o_mob),
      p['Wg'], p['bg'].reshape(1, o_mob),
      p['ln_g'].reshape(1, o_mob), p['ln_b'].reshape(1, o_mob))


def _sage2(x, src, dst, layers):
    n, din = x.shape
    e = src.shape[0]
    h = x
    for ws, wn, b, residual in layers:
        d = h.shape[1]
        dpad = -(-d // 128) * 128
        hp = h if dpad == d else jnp.pad(h, ((0, 0), (0, dpad - d)))
        zer = jnp.zeros((n, dpad), jnp.float32)
        aggp, degp = _seg_sum_sc(n, dpad, e)(hp, src, dst, zer)
        if dpad != d:
            aggp = aggp[:, :, :d]
        h = _sage_dense(h, aggp, degp.reshape(NW, n), ws, wn, b, residual)
    return h


def kernel(mobility_x, mobility_edge_index, edar_x, edar_edge_index,
           edar_muni_mask, params):
    p = params
    mob_src, mob_dst = mobility_edge_index[0], mobility_edge_index[1]
    ed_src, ed_dst = edar_edge_index[0], edar_edge_index[1]

    mob = _sage2(mobility_x, mob_src, mob_dst, [
        (p['mob_Ws1'], p['mob_Wn1'], p['mob_b1'], True),
        (p['mob_Ws2'], p['mob_Wn2'], p['mob_b2'], False),
    ])
    ed = _sage2(edar_x, ed_src, ed_dst, [
        (p['ed_Ws1'], p['ed_Wn1'], p['ed_b1'], False),
        (p['ed_Ws2'], p['ed_Wn2'], p['ed_b2'], False),
    ])
    return _attn_fuse(mob, ed, edar_muni_mask, p)


# speedup vs baseline: 6.1896x; 1.2693x over previous
import functools

import jax
import jax.numpy as jnp
from jax import lax
from jax.experimental import pallas as pl
from jax.experimental.pallas import tpu as pltpu
from jax.experimental.pallas import tpu_sc as plsc

NC = 2
NS = 16
L = 16
NW = NC * NS
CH = 128


@functools.lru_cache(maxsize=None)
def _seg_sum_sc(n_nodes, dim, n_edges):
    assert n_nodes % 8 == 0 and n_edges % CH == 0 and dim % L == 0
    full = (-(-n_nodes // NS) + 7) // 8 * 8
    tail = n_nodes - full * (NS - 1)
    assert 0 < tail <= full
    n_chunks = n_edges // CH
    base_chunks = n_chunks // NW
    rem = n_chunks % NW

    mesh = plsc.VectorSubcoreMesh(core_axis_name="c", subcore_axis_name="s",
                                  num_cores=NC, num_subcores=NS)
    @functools.partial(
        pl.kernel,
        mesh=mesh,
        compiler_params=pltpu.CompilerParams(needs_layout_passes=False),
        out_type=[
            jax.ShapeDtypeStruct((NC, n_nodes, dim), jnp.float32),
            jax.ShapeDtypeStruct((NW * n_nodes,), jnp.float32),
        ],
        scratch_types=[
            pltpu.VMEM((2, CH), jnp.int32),
            pltpu.VMEM((2, CH), jnp.int32),
            pltpu.VMEM((2, CH, dim), jnp.float32),
            pltpu.VMEM((n_nodes,), jnp.float32),
            pltpu.VMEM_SHARED((n_nodes, dim), jnp.float32),
            pltpu.SemaphoreType.DMA((2,)),
            pltpu.SemaphoreType.DMA((2,)),
            pltpu.SemaphoreType.DMA((2,)),
        ],
    )
    def k(x_hbm, src_hbm, dst_hbm, zeros_hbm, agg_out, deg_out,
          srcb, dstb, rows, degv, agg_sh, sem_g, sem_s, sem_d):
        c = lax.axis_index("c")
        s = lax.axis_index("s")
        wid = s * NC + c

        my_n = base_chunks + jnp.where(wid < rem, 1, 0)
        my_first = wid * base_chunks + jnp.minimum(wid, rem)

        zero16 = jnp.zeros((L,), jnp.float32)

        def zdeg(i, _):
            degv[pl.ds(i * L, L)] = zero16
            return 0

        lax.fori_loop(0, n_nodes // L, zdeg, 0)

        r0 = s * full

        @pl.when(s < NS - 1)
        def _():
            pltpu.sync_copy(zeros_hbm.at[pl.ds(r0, full)],
                            agg_sh.at[pl.ds(r0, full)])

        @pl.when(s == NS - 1)
        def _():
            pltpu.sync_copy(zeros_hbm.at[pl.ds(r0, tail)],
                            agg_sh.at[pl.ds(r0, tail)])

        plsc.subcore_barrier()

        ones16 = jnp.ones((L,), jnp.float32)

        @pl.when(my_n > 0)
        def _():
            ebase0 = my_first * CH
            pltpu.sync_copy(src_hbm.at[pl.ds(ebase0, CH)], srcb.at[0])
            pltpu.sync_copy(dst_hbm.at[pl.ds(ebase0, CH)], dstb.at[0])
            pltpu.async_copy(x_hbm.at[srcb.at[0]], rows.at[0], sem_g.at[0])

        def body(i, _):
            b = lax.rem(i, 2)
            nb = 1 - b
            ebase1 = (my_first + i + 1) * CH

            @pl.when(i + 1 < my_n)
            def _():
                pltpu.async_copy(src_hbm.at[pl.ds(ebase1, CH)], srcb.at[nb],
                                 sem_s.at[nb])
                pltpu.async_copy(dst_hbm.at[pl.ds(ebase1, CH)], dstb.at[nb],
                                 sem_d.at[nb])

            pltpu.make_async_copy(x_hbm.at[srcb.at[b]], rows.at[b],
                                  sem_g.at[b]).wait()
            pltpu.sync_copy(rows.at[b], agg_sh.at[dstb.at[b]], add=True)
            for j in range(CH // L):
                idx = dstb[b, pl.ds(j * L, L)]
                plsc.addupdate_scatter(degv, [idx], ones16)

            @pl.when(i + 1 < my_n)
            def _():
                pltpu.make_async_copy(src_hbm.at[pl.ds(ebase1, CH)],
                                      srcb.at[nb], sem_s.at[nb]).wait()
                pltpu.make_async_copy(dst_hbm.at[pl.ds(ebase1, CH)],
                                      dstb.at[nb], sem_d.at[nb]).wait()
                pltpu.async_copy(x_hbm.at[srcb.at[nb]], rows.at[nb],
                                 sem_g.at[nb])
            return 0

        lax.fori_loop(0, my_n, body, 0)

        pltpu.sync_copy(degv, deg_out.at[pl.ds(wid * n_nodes, n_nodes)])
        plsc.subcore_barrier()

        @pl.when(s < NS - 1)
        def _():
            pltpu.sync_copy(agg_sh.at[pl.ds(r0, full)],
                            agg_out.at[c, pl.ds(r0, full)])

        @pl.when(s == NS - 1)
        def _():
            pltpu.sync_copy(agg_sh.at[pl.ds(r0, tail)],
                            agg_out.at[c, pl.ds(r0, tail)])

    return k


def _sage_dense(x, aggp, degp, ws, wn, b, residual):
    n, din = x.shape
    dout = ws.shape[1]
    r = 512
    grid = (pl.cdiv(n, r),)

    def body(x_ref, a_ref, d_ref, ws_ref, wn_ref, b_ref, o_ref):
        agg = a_ref[0] + a_ref[1]
        deg = jnp.sum(d_ref[...], axis=0)[:, None]
        mean = agg / jnp.maximum(deg, 1.0)
        h = jnp.dot(x_ref[...], ws_ref[...], preferred_element_type=jnp.float32)
        h = h + jnp.dot(mean, wn_ref[...], preferred_element_type=jnp.float32)
        h = jnp.maximum(h + b_ref[...], 0.0)
        if residual:
            h = h + x_ref[...]
        o_ref[...] = h

    return pl.pallas_call(
        body,
        grid=grid,
        in_specs=[
            pl.BlockSpec((r, din), lambda i: (i, 0)),
            pl.BlockSpec((NC, r, din), lambda i: (0, i, 0)),
            pl.BlockSpec((NW, r), lambda i: (0, i)),
            pl.BlockSpec((din, dout), lambda i: (0, 0)),
            pl.BlockSpec((din, dout), lambda i: (0, 0)),
            pl.BlockSpec((1, dout), lambda i: (0, 0)),
        ],
        out_specs=pl.BlockSpec((r, dout), lambda i: (i, 0)),
        out_shape=jax.ShapeDtypeStruct((n, dout), jnp.float32),
    )(x, aggp, degp, ws, wn, b.reshape(1, dout))


def _attn_fuse(mob, ed, mask, p):
    n_mob, o_mob = mob.shape
    n_ed, o_ed = ed.shape
    r = 512
    grid = (pl.cdiv(n_mob, r),)

    def body(mob_ref, ed_ref, mask_ref, wq_ref, wk_ref, alpha_ref,
             wf1_ref, bf1_ref, wf2_ref, bf2_ref, wg_ref, bg_ref,
             lng_ref, lnb_ref, o_ref):
        m = mob_ref[...]
        ed_f = ed_ref[...]
        q = jnp.dot(m, wq_ref[...], preferred_element_type=jnp.float32)
        kk = jnp.dot(ed_f, wk_ref[...], preferred_element_type=jnp.float32)
        scores = lax.dot_general(
            q, kk, (((1,), (1,)), ((), ())),
            preferred_element_type=jnp.float32)
        smax = jnp.max(scores, axis=1, keepdims=True)
        e = jnp.exp(scores - smax)
        learned = e / jnp.sum(e, axis=1, keepdims=True)
        msk = mask_ref[...]
        mask_norm = msk / jnp.maximum(jnp.sum(msk, axis=1, keepdims=True), 1e-8)
        g = 1.0 / (1.0 + jnp.exp(-alpha_ref[0, 0]))
        attn = g * mask_norm + (1.0 - g) * learned
        sig = jnp.dot(attn, ed_f, preferred_element_type=jnp.float32)

        wf1 = wf1_ref[...]
        h1 = (jnp.dot(m, wf1[:o_mob, :], preferred_element_type=jnp.float32)
              + jnp.dot(sig, wf1[o_mob:o_mob + o_ed, :],
                        preferred_element_type=jnp.float32) + bf1_ref[...])
        h1 = jnp.maximum(h1, 0.0)
        fused = jnp.dot(h1, wf2_ref[...],
                        preferred_element_type=jnp.float32) + bf2_ref[...]
        wg = wg_ref[...]
        gz = (jnp.dot(m, wg[:o_mob, :], preferred_element_type=jnp.float32)
              + jnp.dot(sig, wg[o_mob:o_mob + o_ed, :],
                        preferred_element_type=jnp.float32) + bg_ref[...])
        gate = 1.0 / (1.0 + jnp.exp(-gz))
        comb = gate * fused + (1.0 - gate) * m
        mu = jnp.mean(comb, axis=-1, keepdims=True)
        var = jnp.mean((comb - mu) ** 2, axis=-1, keepdims=True)
        comb = (comb - mu) / jnp.sqrt(var + 1e-5) * lng_ref[...] + lnb_ref[...]
        nrm = jnp.sqrt(jnp.sum(comb * comb, axis=-1, keepdims=True))
        o_ref[...] = comb / jnp.maximum(nrm, 1e-12)

    o_ed_dim = p['Wk'].shape[1]
    return pl.pallas_call(
        body,
        grid=grid,
        in_specs=[
            pl.BlockSpec((r, o_mob), lambda i: (i, 0)),
            pl.BlockSpec((n_ed, o_ed), lambda i: (0, 0)),
            pl.BlockSpec((r, n_ed), lambda i: (i, 0)),
            pl.BlockSpec(p['Wq'].shape, lambda i: (0, 0)),
            pl.BlockSpec(p['Wk'].shape, lambda i: (0, 0)),
            pl.BlockSpec((1, 1), lambda i: (0, 0)),
            pl.BlockSpec(p['Wf1'].shape, lambda i: (0, 0)),
            pl.BlockSpec((1, o_mob), lambda i: (0, 0)),
            pl.BlockSpec(p['Wf2'].shape, lambda i: (0, 0)),
            pl.BlockSpec((1, o_mob), lambda i: (0, 0)),
            pl.BlockSpec(p['Wg'].shape, lambda i: (0, 0)),
            pl.BlockSpec((1, o_mob), lambda i: (0, 0)),
            pl.BlockSpec((1, o_mob), lambda i: (0, 0)),
            pl.BlockSpec((1, o_mob), lambda i: (0, 0)),
        ],
        out_specs=pl.BlockSpec((r, o_mob), lambda i: (i, 0)),
        out_shape=jax.ShapeDtypeStruct((n_mob, o_mob), jnp.float32),
    )(mob, ed, mask,
      p['Wq'], p['Wk'], p['alpha'].reshape(1, 1),
      p['Wf1'], p['bf1'].reshape(1, o_mob), p['Wf2'], p['bf2'].reshape(1, o_mob),
      p['Wg'], p['bg'].reshape(1, o_mob),
      p['ln_g'].reshape(1, o_mob), p['ln_b'].reshape(1, o_mob))


def _sage2(x, src, dst, layers):
    n, din = x.shape
    e = src.shape[0]
    h = x
    for ws, wn, b, residual in layers:
        d = h.shape[1]
        dpad = -(-d // 128) * 128
        hp = h if dpad == d else jnp.pad(h, ((0, 0), (0, dpad - d)))
        zer = jnp.zeros((n, dpad), jnp.float32)
        aggp, degp = _seg_sum_sc(n, dpad, e)(hp, src, dst, zer)
        if dpad != d:
            aggp = aggp[:, :, :d]
        h = _sage_dense(h, aggp, degp.reshape(NW, n), ws, wn, b, residual)
    return h


def kernel(mobility_x, mobility_edge_index, edar_x, edar_edge_index,
           edar_muni_mask, params):
    p = params
    mob_src, mob_dst = mobility_edge_index[0], mobility_edge_index[1]
    ed_src, ed_dst = edar_edge_index[0], edar_edge_index[1]

    mob = _sage2(mobility_x, mob_src, mob_dst, [
        (p['mob_Ws1'], p['mob_Wn1'], p['mob_b1'], True),
        (p['mob_Ws2'], p['mob_Wn2'], p['mob_b2'], False),
    ])
    ed = _sage2(edar_x, ed_src, ed_dst, [
        (p['ed_Ws1'], p['ed_Wn1'], p['ed_b1'], False),
        (p['ed_Ws2'], p['ed_Wn2'], p['ed_b2'], False),
    ])
    return _attn_fuse(mob, ed, edar_muni_mask, p)


# speedup vs baseline: 7.4068x; 1.1967x over previous
import functools

import jax
import jax.numpy as jnp
from jax import lax
from jax.experimental import pallas as pl
from jax.experimental.pallas import tpu as pltpu
from jax.experimental.pallas import tpu_sc as plsc

NC = 2
NS = 16
L = 16
NW = NC * NS
CH = 128


@functools.lru_cache(maxsize=None)
def _seg_sum_sc(n_nodes, dim, n_edges):
    assert n_nodes % 8 == 0 and n_edges % CH == 0 and dim % L == 0
    full = (-(-n_nodes // NS) + 7) // 8 * 8
    tail = n_nodes - full * (NS - 1)
    assert 0 < tail <= full
    n_chunks = n_edges // CH
    base_chunks = n_chunks // NW
    rem = n_chunks % NW

    mesh = plsc.VectorSubcoreMesh(core_axis_name="c", subcore_axis_name="s",
                                  num_cores=NC, num_subcores=NS)
    @functools.partial(
        pl.kernel,
        mesh=mesh,
        compiler_params=pltpu.CompilerParams(needs_layout_passes=False),
        out_type=[
            jax.ShapeDtypeStruct((NC, n_nodes, dim), jnp.float32),
            jax.ShapeDtypeStruct((NW * n_nodes,), jnp.float32),
        ],
        scratch_types=[
            pltpu.VMEM((2, CH), jnp.int32),
            pltpu.VMEM((3, CH), jnp.int32),
            pltpu.VMEM((2, CH, dim), jnp.float32),
            pltpu.VMEM((n_nodes,), jnp.float32),
            pltpu.VMEM_SHARED((n_nodes, dim), jnp.float32),
            pltpu.SemaphoreType.DMA((2,)),
            pltpu.SemaphoreType.DMA((2,)),
            pltpu.SemaphoreType.DMA((3,)),
            pltpu.SemaphoreType.DMA((2,)),
        ],
    )
    def k(x_hbm, src_hbm, dst_hbm, zeros_hbm, agg_out, deg_out,
          srcb, dstb, rows, degv, agg_sh, sem_g, sem_s, sem_d, sem_sc):
        c = lax.axis_index("c")
        s = lax.axis_index("s")
        wid = s * NC + c

        my_n = base_chunks + jnp.where(wid < rem, 1, 0)
        my_first = wid * base_chunks + jnp.minimum(wid, rem)

        zero16 = jnp.zeros((L,), jnp.float32)

        def zdeg(i, _):
            degv[pl.ds(i * L, L)] = zero16
            return 0

        lax.fori_loop(0, n_nodes // L, zdeg, 0)

        r0 = s * full

        @pl.when(s < NS - 1)
        def _():
            pltpu.sync_copy(zeros_hbm.at[pl.ds(r0, full)],
                            agg_sh.at[pl.ds(r0, full)])

        @pl.when(s == NS - 1)
        def _():
            pltpu.sync_copy(zeros_hbm.at[pl.ds(r0, tail)],
                            agg_sh.at[pl.ds(r0, tail)])

        plsc.subcore_barrier()

        ones16 = jnp.ones((L,), jnp.float32)

        @pl.when(my_n > 0)
        def _():
            ebase0 = my_first * CH
            pltpu.sync_copy(src_hbm.at[pl.ds(ebase0, CH)], srcb.at[0])
            pltpu.sync_copy(dst_hbm.at[pl.ds(ebase0, CH)], dstb.at[0])
            pltpu.async_copy(x_hbm.at[srcb.at[0]], rows.at[0], sem_g.at[0])

        def body(i, _):
            b = lax.rem(i, 2)
            nb = 1 - b
            d0 = lax.rem(i, 3)
            d1 = lax.rem(i + 1, 3)
            ebase1 = (my_first + i + 1) * CH

            @pl.when(i + 1 < my_n)
            def _():
                pltpu.async_copy(src_hbm.at[pl.ds(ebase1, CH)], srcb.at[nb],
                                 sem_s.at[nb])
                pltpu.async_copy(dst_hbm.at[pl.ds(ebase1, CH)], dstb.at[d1],
                                 sem_d.at[d1])

            pltpu.make_async_copy(x_hbm.at[srcb.at[b]], rows.at[b],
                                  sem_g.at[b]).wait()
            pltpu.async_copy(rows.at[b], agg_sh.at[dstb.at[d0]], sem_sc.at[b],
                             add=True)
            for j in range(CH // L):
                idx = dstb[d0, pl.ds(j * L, L)]
                plsc.addupdate_scatter(degv, [idx], ones16)

            @pl.when(i + 1 < my_n)
            def _():
                pltpu.make_async_copy(src_hbm.at[pl.ds(ebase1, CH)],
                                      srcb.at[nb], sem_s.at[nb]).wait()
                pltpu.make_async_copy(dst_hbm.at[pl.ds(ebase1, CH)],
                                      dstb.at[d1], sem_d.at[d1]).wait()

                @pl.when(i >= 1)
                def _():
                    pltpu.make_async_copy(rows.at[nb],
                                          agg_sh.at[dstb.at[d1]],
                                          sem_sc.at[nb]).wait()

                pltpu.async_copy(x_hbm.at[srcb.at[nb]], rows.at[nb],
                                 sem_g.at[nb])
            return 0

        lax.fori_loop(0, my_n, body, 0)

        @pl.when(my_n >= 1)
        def _():
            bb = lax.rem(my_n - 1, 2)
            pltpu.make_async_copy(rows.at[bb], agg_sh.at[dstb.at[0]],
                                  sem_sc.at[bb]).wait()

        @pl.when(my_n >= 2)
        def _():
            bb = lax.rem(my_n - 2, 2)
            pltpu.make_async_copy(rows.at[bb], agg_sh.at[dstb.at[0]],
                                  sem_sc.at[bb]).wait()

        pltpu.sync_copy(degv, deg_out.at[pl.ds(wid * n_nodes, n_nodes)])
        plsc.subcore_barrier()

        @pl.when(s < NS - 1)
        def _():
            pltpu.sync_copy(agg_sh.at[pl.ds(r0, full)],
                            agg_out.at[c, pl.ds(r0, full)])

        @pl.when(s == NS - 1)
        def _():
            pltpu.sync_copy(agg_sh.at[pl.ds(r0, tail)],
                            agg_out.at[c, pl.ds(r0, tail)])

    return k


def _sage_dense(x, aggp, degp, ws, wn, b, residual):
    n, din = x.shape
    dout = ws.shape[1]
    r = 512
    grid = (pl.cdiv(n, r),)

    def body(x_ref, a_ref, d_ref, ws_ref, wn_ref, b_ref, o_ref):
        agg = a_ref[0] + a_ref[1]
        deg = jnp.sum(d_ref[...], axis=0)[:, None]
        mean = agg / jnp.maximum(deg, 1.0)
        h = jnp.dot(x_ref[...], ws_ref[...], preferred_element_type=jnp.float32)
        h = h + jnp.dot(mean, wn_ref[...], preferred_element_type=jnp.float32)
        h = jnp.maximum(h + b_ref[...], 0.0)
        if residual:
            h = h + x_ref[...]
        o_ref[...] = h

    return pl.pallas_call(
        body,
        grid=grid,
        in_specs=[
            pl.BlockSpec((r, din), lambda i: (i, 0)),
            pl.BlockSpec((NC, r, din), lambda i: (0, i, 0)),
            pl.BlockSpec((NW, r), lambda i: (0, i)),
            pl.BlockSpec((din, dout), lambda i: (0, 0)),
            pl.BlockSpec((din, dout), lambda i: (0, 0)),
            pl.BlockSpec((1, dout), lambda i: (0, 0)),
        ],
        out_specs=pl.BlockSpec((r, dout), lambda i: (i, 0)),
        out_shape=jax.ShapeDtypeStruct((n, dout), jnp.float32),
    )(x, aggp, degp, ws, wn, b.reshape(1, dout))


def _attn_fuse(mob, ed, mask, p):
    n_mob, o_mob = mob.shape
    n_ed, o_ed = ed.shape
    r = 512
    grid = (pl.cdiv(n_mob, r),)

    def body(mob_ref, ed_ref, mask_ref, wq_ref, wk_ref, alpha_ref,
             wf1_ref, bf1_ref, wf2_ref, bf2_ref, wg_ref, bg_ref,
             lng_ref, lnb_ref, o_ref):
        m = mob_ref[...]
        ed_f = ed_ref[...]
        q = jnp.dot(m, wq_ref[...], preferred_element_type=jnp.float32)
        kk = jnp.dot(ed_f, wk_ref[...], preferred_element_type=jnp.float32)
        scores = lax.dot_general(
            q, kk, (((1,), (1,)), ((), ())),
            preferred_element_type=jnp.float32)
        smax = jnp.max(scores, axis=1, keepdims=True)
        e = jnp.exp(scores - smax)
        learned = e / jnp.sum(e, axis=1, keepdims=True)
        msk = mask_ref[...]
        mask_norm = msk / jnp.maximum(jnp.sum(msk, axis=1, keepdims=True), 1e-8)
        g = 1.0 / (1.0 + jnp.exp(-alpha_ref[0, 0]))
        attn = g * mask_norm + (1.0 - g) * learned
        sig = jnp.dot(attn, ed_f, preferred_element_type=jnp.float32)

        wf1 = wf1_ref[...]
        h1 = (jnp.dot(m, wf1[:o_mob, :], preferred_element_type=jnp.float32)
              + jnp.dot(sig, wf1[o_mob:o_mob + o_ed, :],
                        preferred_element_type=jnp.float32) + bf1_ref[...])
        h1 = jnp.maximum(h1, 0.0)
        fused = jnp.dot(h1, wf2_ref[...],
                        preferred_element_type=jnp.float32) + bf2_ref[...]
        wg = wg_ref[...]
        gz = (jnp.dot(m, wg[:o_mob, :], preferred_element_type=jnp.float32)
              + jnp.dot(sig, wg[o_mob:o_mob + o_ed, :],
                        preferred_element_type=jnp.float32) + bg_ref[...])
        gate = 1.0 / (1.0 + jnp.exp(-gz))
        comb = gate * fused + (1.0 - gate) * m
        mu = jnp.mean(comb, axis=-1, keepdims=True)
        var = jnp.mean((comb - mu) ** 2, axis=-1, keepdims=True)
        comb = (comb - mu) / jnp.sqrt(var + 1e-5) * lng_ref[...] + lnb_ref[...]
        nrm = jnp.sqrt(jnp.sum(comb * comb, axis=-1, keepdims=True))
        o_ref[...] = comb / jnp.maximum(nrm, 1e-12)

    o_ed_dim = p['Wk'].shape[1]
    return pl.pallas_call(
        body,
        grid=grid,
        in_specs=[
            pl.BlockSpec((r, o_mob), lambda i: (i, 0)),
            pl.BlockSpec((n_ed, o_ed), lambda i: (0, 0)),
            pl.BlockSpec((r, n_ed), lambda i: (i, 0)),
            pl.BlockSpec(p['Wq'].shape, lambda i: (0, 0)),
            pl.BlockSpec(p['Wk'].shape, lambda i: (0, 0)),
            pl.BlockSpec((1, 1), lambda i: (0, 0)),
            pl.BlockSpec(p['Wf1'].shape, lambda i: (0, 0)),
            pl.BlockSpec((1, o_mob), lambda i: (0, 0)),
            pl.BlockSpec(p['Wf2'].shape, lambda i: (0, 0)),
            pl.BlockSpec((1, o_mob), lambda i: (0, 0)),
            pl.BlockSpec(p['Wg'].shape, lambda i: (0, 0)),
            pl.BlockSpec((1, o_mob), lambda i: (0, 0)),
            pl.BlockSpec((1, o_mob), lambda i: (0, 0)),
            pl.BlockSpec((1, o_mob), lambda i: (0, 0)),
        ],
        out_specs=pl.BlockSpec((r, o_mob), lambda i: (i, 0)),
        out_shape=jax.ShapeDtypeStruct((n_mob, o_mob), jnp.float32),
    )(mob, ed, mask,
      p['Wq'], p['Wk'], p['alpha'].reshape(1, 1),
      p['Wf1'], p['bf1'].reshape(1, o_mob), p['Wf2'], p['bf2'].reshape(1, o_mob),
      p['Wg'], p['bg'].reshape(1, o_mob),
      p['ln_g'].reshape(1, o_mob), p['ln_b'].reshape(1, o_mob))


def _sage2(x, src, dst, layers):
    n, din = x.shape
    e = src.shape[0]
    h = x
    for ws, wn, b, residual in layers:
        d = h.shape[1]
        dpad = -(-d // 128) * 128
        hp = h if dpad == d else jnp.pad(h, ((0, 0), (0, dpad - d)))
        zer = jnp.zeros((n, dpad), jnp.float32)
        aggp, degp = _seg_sum_sc(n, dpad, e)(hp, src, dst, zer)
        if dpad != d:
            aggp = aggp[:, :, :d]
        h = _sage_dense(h, aggp, degp.reshape(NW, n), ws, wn, b, residual)
    return h


def kernel(mobility_x, mobility_edge_index, edar_x, edar_edge_index,
           edar_muni_mask, params):
    p = params
    mob_src, mob_dst = mobility_edge_index[0], mobility_edge_index[1]
    ed_src, ed_dst = edar_edge_index[0], edar_edge_index[1]

    mob = _sage2(mobility_x, mob_src, mob_dst, [
        (p['mob_Ws1'], p['mob_Wn1'], p['mob_b1'], True),
        (p['mob_Ws2'], p['mob_Wn2'], p['mob_b2'], False),
    ])
    ed = _sage2(edar_x, ed_src, ed_dst, [
        (p['ed_Ws1'], p['ed_Wn1'], p['ed_b1'], False),
        (p['ed_Ws2'], p['ed_Wn2'], p['ed_b2'], False),
    ])
    return _attn_fuse(mob, ed, edar_muni_mask, p)


# speedup vs baseline: 7.6709x; 1.0357x over previous
import functools

import jax
import jax.numpy as jnp
from jax import lax
from jax.experimental import pallas as pl
from jax.experimental.pallas import tpu as pltpu
from jax.experimental.pallas import tpu_sc as plsc

NC = 2
NS = 16
L = 16
NW = NC * NS
CH = 128
NB = 2
NE = 4
P = 2


@functools.lru_cache(maxsize=None)
def _seg_sum_sc(n_nodes, dim, n_edges, with_deg=True):
    assert n_nodes % 8 == 0 and n_edges % CH == 0 and dim % L == 0
    full = (-(-n_nodes // NS) + 7) // 8 * 8
    tail = n_nodes - full * (NS - 1)
    assert 0 < tail <= full
    n_chunks = n_edges // CH
    base_chunks = n_chunks // NW
    rem = n_chunks % NW

    mesh = plsc.VectorSubcoreMesh(core_axis_name="c", subcore_axis_name="s",
                                  num_cores=NC, num_subcores=NS)
    agg_t = jax.ShapeDtypeStruct((NC, n_nodes, dim), jnp.float32)
    out_type = ([agg_t, jax.ShapeDtypeStruct((NW * n_nodes,), jnp.float32)]
                if with_deg else agg_t)
    scratch = [
        pltpu.VMEM((NE, 2, CH), jnp.int32),
        pltpu.VMEM((NB, CH, dim), jnp.float32),
        pltpu.VMEM_SHARED((n_nodes, dim), jnp.float32),
        pltpu.SemaphoreType.DMA((NE,)),
        pltpu.SemaphoreType.DMA((NB,)),
        pltpu.SemaphoreType.DMA((NB,)),
    ]
    if with_deg:
        scratch.append(pltpu.VMEM((n_nodes,), jnp.float32))

    def impl(x_hbm, ei_hbm, zeros_hbm, agg_out,
             eib, rows, agg_sh, sem_e, sem_g, sem_sc, deg_out, degv):
        c = lax.axis_index("c")
        s = lax.axis_index("s")
        wid = s * NC + c

        my_n = base_chunks + jnp.where(wid < rem, 1, 0)
        my_first = wid * base_chunks + jnp.minimum(wid, rem)

        if with_deg:
            zero16 = jnp.zeros((L,), jnp.float32)

            def zdeg(i, _):
                degv[pl.ds(i * L, L)] = zero16
                return 0

            lax.fori_loop(0, n_nodes // L, zdeg, 0)

        r0 = s * full

        @pl.when(s < NS - 1)
        def _():
            pltpu.sync_copy(zeros_hbm.at[pl.ds(r0, full)],
                            agg_sh.at[pl.ds(r0, full)])

        @pl.when(s == NS - 1)
        def _():
            pltpu.sync_copy(zeros_hbm.at[pl.ds(r0, tail)],
                            agg_sh.at[pl.ds(r0, tail)])

        plsc.subcore_barrier()

        ones16 = jnp.ones((L,), jnp.float32)

        def fetch_idx(j):
            pltpu.async_copy(
                ei_hbm.at[pl.ds(0, 2), pl.ds((my_first + j) * CH, CH)],
                eib.at[lax.rem(j, NE)], sem_e.at[lax.rem(j, NE)])

        def wait_idx(j):
            pltpu.make_async_copy(
                ei_hbm.at[pl.ds(0, 2), pl.ds((my_first + j) * CH, CH)],
                eib.at[lax.rem(j, NE)], sem_e.at[lax.rem(j, NE)]).wait()

        def fire_gather(j):
            pltpu.async_copy(x_hbm.at[eib.at[lax.rem(j, NE), 0]],
                             rows.at[lax.rem(j, NB)],
                             sem_g.at[lax.rem(j, NB)])

        def wait_gather(j):
            pltpu.make_async_copy(x_hbm.at[eib.at[lax.rem(j, NE), 0]],
                                  rows.at[lax.rem(j, NB)],
                                  sem_g.at[lax.rem(j, NB)]).wait()

        def wait_scatter_slot(slot):
            pltpu.make_async_copy(rows.at[slot], agg_sh.at[eib.at[0, 1]],
                                  sem_sc.at[slot]).wait()

        for kk in range(P):
            @pl.when(kk < my_n)
            def _(kk=kk):
                fetch_idx(kk)

        for kk in range(P - 1):
            @pl.when(kk < my_n)
            def _(kk=kk):
                wait_idx(kk)
                fire_gather(kk)

        def body(i, _):
            @pl.when(i + P < my_n)
            def _():
                fetch_idx(i + P)

            wait_gather(i)
            e0 = lax.rem(i, NE)
            pltpu.async_copy(rows.at[lax.rem(i, NB)],
                             agg_sh.at[eib.at[e0, 1]],
                             sem_sc.at[lax.rem(i, NB)], add=True)
            if with_deg:
                for j in range(CH // L):
                    idx = eib[e0, 1, pl.ds(j * L, L)]
                    plsc.addupdate_scatter(degv, [idx], ones16)

            @pl.when(i + P - 1 < my_n)
            def _():
                wait_idx(i + P - 1)

                @pl.when(i + P - 1 >= NB)
                def _():
                    wait_scatter_slot(lax.rem(i + P - 1, NB))

                fire_gather(i + P - 1)
            return 0

        lax.fori_loop(0, my_n, body, 0)

        for kk in range(1, NB + 1):
            @pl.when(my_n >= kk)
            def _(kk=kk):
                wait_scatter_slot(lax.rem(my_n - kk + 2 * NB, NB))

        if with_deg:
            pltpu.sync_copy(degv, deg_out.at[pl.ds(wid * n_nodes, n_nodes)])
        plsc.subcore_barrier()

        @pl.when(s < NS - 1)
        def _():
            pltpu.sync_copy(agg_sh.at[pl.ds(r0, full)],
                            agg_out.at[c, pl.ds(r0, full)])

        @pl.when(s == NS - 1)
        def _():
            pltpu.sync_copy(agg_sh.at[pl.ds(r0, tail)],
                            agg_out.at[c, pl.ds(r0, tail)])

    kw = dict(mesh=mesh, out_type=out_type, scratch_types=scratch,
              compiler_params=pltpu.CompilerParams(needs_layout_passes=False))
    if with_deg:
        @functools.partial(pl.kernel, **kw)
        def k(x_hbm, ei_hbm, zeros_hbm, agg_out, deg_out,
              eib, rows, agg_sh, sem_e, sem_g, sem_sc, degv):
            impl(x_hbm, ei_hbm, zeros_hbm, agg_out,
                 eib, rows, agg_sh, sem_e, sem_g, sem_sc, deg_out, degv)
    else:
        @functools.partial(pl.kernel, **kw)
        def k(x_hbm, ei_hbm, zeros_hbm, agg_out,
              eib, rows, agg_sh, sem_e, sem_g, sem_sc):
            impl(x_hbm, ei_hbm, zeros_hbm, agg_out,
                 eib, rows, agg_sh, sem_e, sem_g, sem_sc, None, None)

    return k


def _sage_dense(x, aggp, degp, ws, wn, b, residual):
    n, din = x.shape
    dout = ws.shape[1]
    r = 512
    grid = (pl.cdiv(n, r),)

    def body(x_ref, a_ref, d_ref, ws_ref, wn_ref, b_ref, o_ref):
        agg = a_ref[0] + a_ref[1]
        deg = jnp.sum(d_ref[...], axis=0)[:, None]
        mean = agg / jnp.maximum(deg, 1.0)
        h = jnp.dot(x_ref[...], ws_ref[...], preferred_element_type=jnp.float32)
        h = h + jnp.dot(mean, wn_ref[...], preferred_element_type=jnp.float32)
        h = jnp.maximum(h + b_ref[...], 0.0)
        if residual:
            h = h + x_ref[...]
        o_ref[...] = h

    return pl.pallas_call(
        body,
        grid=grid,
        in_specs=[
            pl.BlockSpec((r, din), lambda i: (i, 0)),
            pl.BlockSpec((NC, r, din), lambda i: (0, i, 0)),
            pl.BlockSpec((NW, r), lambda i: (0, i)),
            pl.BlockSpec((din, dout), lambda i: (0, 0)),
            pl.BlockSpec((din, dout), lambda i: (0, 0)),
            pl.BlockSpec((1, dout), lambda i: (0, 0)),
        ],
        out_specs=pl.BlockSpec((r, dout), lambda i: (i, 0)),
        out_shape=jax.ShapeDtypeStruct((n, dout), jnp.float32),
    )(x, aggp, degp, ws, wn, b.reshape(1, dout))


def _attn_fuse(mob, ed, mask, p):
    n_mob, o_mob = mob.shape
    n_ed, o_ed = ed.shape
    r = 512
    grid = (pl.cdiv(n_mob, r),)

    def body(mob_ref, ed_ref, mask_ref, wq_ref, wk_ref, alpha_ref,
             wf1_ref, bf1_ref, wf2_ref, bf2_ref, wg_ref, bg_ref,
             lng_ref, lnb_ref, o_ref):
        m = mob_ref[...]
        ed_f = ed_ref[...]
        q = jnp.dot(m, wq_ref[...], preferred_element_type=jnp.float32)
        kk = jnp.dot(ed_f, wk_ref[...], preferred_element_type=jnp.float32)
        scores = lax.dot_general(
            q, kk, (((1,), (1,)), ((), ())),
            preferred_element_type=jnp.float32)
        smax = jnp.max(scores, axis=1, keepdims=True)
        e = jnp.exp(scores - smax)
        learned = e / jnp.sum(e, axis=1, keepdims=True)
        msk = mask_ref[...]
        mask_norm = msk / jnp.maximum(jnp.sum(msk, axis=1, keepdims=True), 1e-8)
        g = 1.0 / (1.0 + jnp.exp(-alpha_ref[0, 0]))
        attn = g * mask_norm + (1.0 - g) * learned
        sig = jnp.dot(attn, ed_f, preferred_element_type=jnp.float32)

        wf1 = wf1_ref[...]
        h1 = (jnp.dot(m, wf1[:o_mob, :], preferred_element_type=jnp.float32)
              + jnp.dot(sig, wf1[o_mob:o_mob + o_ed, :],
                        preferred_element_type=jnp.float32) + bf1_ref[...])
        h1 = jnp.maximum(h1, 0.0)
        fused = jnp.dot(h1, wf2_ref[...],
                        preferred_element_type=jnp.float32) + bf2_ref[...]
        wg = wg_ref[...]
        gz = (jnp.dot(m, wg[:o_mob, :], preferred_element_type=jnp.float32)
              + jnp.dot(sig, wg[o_mob:o_mob + o_ed, :],
                        preferred_element_type=jnp.float32) + bg_ref[...])
        gate = 1.0 / (1.0 + jnp.exp(-gz))
        comb = gate * fused + (1.0 - gate) * m
        mu = jnp.mean(comb, axis=-1, keepdims=True)
        var = jnp.mean((comb - mu) ** 2, axis=-1, keepdims=True)
        comb = (comb - mu) / jnp.sqrt(var + 1e-5) * lng_ref[...] + lnb_ref[...]
        nrm = jnp.sqrt(jnp.sum(comb * comb, axis=-1, keepdims=True))
        o_ref[...] = comb / jnp.maximum(nrm, 1e-12)

    o_ed_dim = p['Wk'].shape[1]
    return pl.pallas_call(
        body,
        grid=grid,
        in_specs=[
            pl.BlockSpec((r, o_mob), lambda i: (i, 0)),
            pl.BlockSpec((n_ed, o_ed), lambda i: (0, 0)),
            pl.BlockSpec((r, n_ed), lambda i: (i, 0)),
            pl.BlockSpec(p['Wq'].shape, lambda i: (0, 0)),
            pl.BlockSpec(p['Wk'].shape, lambda i: (0, 0)),
            pl.BlockSpec((1, 1), lambda i: (0, 0)),
            pl.BlockSpec(p['Wf1'].shape, lambda i: (0, 0)),
            pl.BlockSpec((1, o_mob), lambda i: (0, 0)),
            pl.BlockSpec(p['Wf2'].shape, lambda i: (0, 0)),
            pl.BlockSpec((1, o_mob), lambda i: (0, 0)),
            pl.BlockSpec(p['Wg'].shape, lambda i: (0, 0)),
            pl.BlockSpec((1, o_mob), lambda i: (0, 0)),
            pl.BlockSpec((1, o_mob), lambda i: (0, 0)),
            pl.BlockSpec((1, o_mob), lambda i: (0, 0)),
        ],
        out_specs=pl.BlockSpec((r, o_mob), lambda i: (i, 0)),
        out_shape=jax.ShapeDtypeStruct((n_mob, o_mob), jnp.float32),
    )(mob, ed, mask,
      p['Wq'], p['Wk'], p['alpha'].reshape(1, 1),
      p['Wf1'], p['bf1'].reshape(1, o_mob), p['Wf2'], p['bf2'].reshape(1, o_mob),
      p['Wg'], p['bg'].reshape(1, o_mob),
      p['ln_g'].reshape(1, o_mob), p['ln_b'].reshape(1, o_mob))


def _sage2(x, ei, layers):
    n, din = x.shape
    e = ei.shape[1]
    h = x
    degp = None
    for ws, wn, b, residual in layers:
        d = h.shape[1]
        dpad = -(-d // 128) * 128
        hp = h if dpad == d else jnp.pad(h, ((0, 0), (0, dpad - d)))
        zer = jnp.zeros((n, dpad), jnp.float32)
        if degp is None:
            aggp, degp = _seg_sum_sc(n, dpad, e, True)(hp, ei, zer)
            degp = degp.reshape(NW, n)
        else:
            aggp = _seg_sum_sc(n, dpad, e, False)(hp, ei, zer)
        if dpad != d:
            aggp = aggp[:, :, :d]
        h = _sage_dense(h, aggp, degp, ws, wn, b, residual)
    return h


def kernel(mobility_x, mobility_edge_index, edar_x, edar_edge_index,
           edar_muni_mask, params):
    p = params

    mob = _sage2(mobility_x, mobility_edge_index, [
        (p['mob_Ws1'], p['mob_Wn1'], p['mob_b1'], True),
        (p['mob_Ws2'], p['mob_Wn2'], p['mob_b2'], False),
    ])
    ed = _sage2(edar_x, edar_edge_index, [
        (p['ed_Ws1'], p['ed_Wn1'], p['ed_b1'], False),
        (p['ed_Ws2'], p['ed_Wn2'], p['ed_b2'], False),
    ])
    return _attn_fuse(mob, ed, edar_muni_mask, p)


# speedup vs baseline: 7.8196x; 1.0194x over previous
import functools

import jax
import jax.numpy as jnp
from jax import lax
from jax.experimental import pallas as pl
from jax.experimental.pallas import tpu as pltpu
from jax.experimental.pallas import tpu_sc as plsc

NC = 2
NS = 16
L = 16
NW = NC * NS
CH = 128
NB = 2
NE = 4
P = 2


@functools.lru_cache(maxsize=None)
def _seg_sum_sc(n_nodes, dim, n_edges, with_deg=True):
    assert n_nodes % 8 == 0 and n_edges % CH == 0 and dim % L == 0
    full = (-(-n_nodes // NS) + 7) // 8 * 8
    tail = n_nodes - full * (NS - 1)
    assert 0 < tail <= full
    n_chunks = n_edges // CH
    base_chunks = n_chunks // NW
    rem = n_chunks % NW

    mesh = plsc.VectorSubcoreMesh(core_axis_name="c", subcore_axis_name="s",
                                  num_cores=NC, num_subcores=NS)
    agg_t = jax.ShapeDtypeStruct((NC, n_nodes, dim), jnp.float32)
    out_type = ([agg_t, jax.ShapeDtypeStruct((NW * n_nodes,), jnp.float32)]
                if with_deg else agg_t)
    scratch = [
        pltpu.VMEM((NE, 2, CH), jnp.int32),
        pltpu.VMEM((NB, CH, dim), jnp.float32),
        pltpu.VMEM_SHARED((n_nodes, dim), jnp.float32),
        pltpu.SemaphoreType.DMA((NE,)),
        pltpu.SemaphoreType.DMA((NB,)),
        pltpu.SemaphoreType.DMA((NB,)),
    ]
    if with_deg:
        scratch.append(pltpu.VMEM((n_nodes,), jnp.float32))

    def impl(x_hbm, ei_hbm, zeros_hbm, agg_out,
             eib, rows, agg_sh, sem_e, sem_g, sem_sc, deg_out, degv):
        c = lax.axis_index("c")
        s = lax.axis_index("s")
        wid = s * NC + c

        my_n = base_chunks + jnp.where(wid < rem, 1, 0)
        my_first = wid * base_chunks + jnp.minimum(wid, rem)

        if with_deg:
            zero16 = jnp.zeros((L,), jnp.float32)

            def zdeg(i, _):
                degv[pl.ds(i * L, L)] = zero16
                return 0

            lax.fori_loop(0, n_nodes // L, zdeg, 0)

        r0 = s * full

        @pl.when(s < NS - 1)
        def _():
            pltpu.sync_copy(zeros_hbm.at[pl.ds(r0, full)],
                            agg_sh.at[pl.ds(r0, full)])

        @pl.when(s == NS - 1)
        def _():
            pltpu.sync_copy(zeros_hbm.at[pl.ds(r0, tail)],
                            agg_sh.at[pl.ds(r0, tail)])

        plsc.subcore_barrier()

        ones16 = jnp.ones((L,), jnp.float32)

        def fetch_idx(j):
            pltpu.async_copy(
                ei_hbm.at[pl.ds(0, 2), pl.ds((my_first + j) * CH, CH)],
                eib.at[lax.rem(j, NE)], sem_e.at[lax.rem(j, NE)])

        def wait_idx(j):
            pltpu.make_async_copy(
                ei_hbm.at[pl.ds(0, 2), pl.ds((my_first + j) * CH, CH)],
                eib.at[lax.rem(j, NE)], sem_e.at[lax.rem(j, NE)]).wait()

        def fire_gather(j):
            pltpu.async_copy(x_hbm.at[eib.at[lax.rem(j, NE), 0]],
                             rows.at[lax.rem(j, NB)],
                             sem_g.at[lax.rem(j, NB)])

        def wait_gather(j):
            pltpu.make_async_copy(x_hbm.at[eib.at[lax.rem(j, NE), 0]],
                                  rows.at[lax.rem(j, NB)],
                                  sem_g.at[lax.rem(j, NB)]).wait()

        def wait_scatter_slot(slot):
            pltpu.make_async_copy(rows.at[slot], agg_sh.at[eib.at[0, 1]],
                                  sem_sc.at[slot]).wait()

        for kk in range(P):
            @pl.when(kk < my_n)
            def _(kk=kk):
                fetch_idx(kk)

        for kk in range(P - 1):
            @pl.when(kk < my_n)
            def _(kk=kk):
                wait_idx(kk)
                fire_gather(kk)

        def body(i, _):
            @pl.when(i + P < my_n)
            def _():
                fetch_idx(i + P)

            wait_gather(i)
            e0 = lax.rem(i, NE)
            pltpu.async_copy(rows.at[lax.rem(i, NB)],
                             agg_sh.at[eib.at[e0, 1]],
                             sem_sc.at[lax.rem(i, NB)], add=True)
            if with_deg:
                for j in range(CH // L):
                    idx = eib[e0, 1, pl.ds(j * L, L)]
                    plsc.addupdate_scatter(degv, [idx], ones16)

            @pl.when(i + P - 1 < my_n)
            def _():
                wait_idx(i + P - 1)

                @pl.when(i + P - 1 >= NB)
                def _():
                    wait_scatter_slot(lax.rem(i + P - 1, NB))

                fire_gather(i + P - 1)
            return 0

        lax.fori_loop(0, my_n, body, 0)

        for kk in range(1, NB + 1):
            @pl.when(my_n >= kk)
            def _(kk=kk):
                wait_scatter_slot(lax.rem(my_n - kk + 2 * NB, NB))

        if with_deg:
            pltpu.sync_copy(degv, deg_out.at[pl.ds(wid * n_nodes, n_nodes)])
        plsc.subcore_barrier()

        @pl.when(s < NS - 1)
        def _():
            pltpu.sync_copy(agg_sh.at[pl.ds(r0, full)],
                            agg_out.at[c, pl.ds(r0, full)])

        @pl.when(s == NS - 1)
        def _():
            pltpu.sync_copy(agg_sh.at[pl.ds(r0, tail)],
                            agg_out.at[c, pl.ds(r0, tail)])

    kw = dict(mesh=mesh, out_type=out_type, scratch_types=scratch,
              compiler_params=pltpu.CompilerParams(needs_layout_passes=False))
    if with_deg:
        @functools.partial(pl.kernel, **kw)
        def k(x_hbm, ei_hbm, zeros_hbm, agg_out, deg_out,
              eib, rows, agg_sh, sem_e, sem_g, sem_sc, degv):
            impl(x_hbm, ei_hbm, zeros_hbm, agg_out,
                 eib, rows, agg_sh, sem_e, sem_g, sem_sc, deg_out, degv)
    else:
        @functools.partial(pl.kernel, **kw)
        def k(x_hbm, ei_hbm, zeros_hbm, agg_out,
              eib, rows, agg_sh, sem_e, sem_g, sem_sc):
            impl(x_hbm, ei_hbm, zeros_hbm, agg_out,
                 eib, rows, agg_sh, sem_e, sem_g, sem_sc, None, None)

    return k


def _sage_dense(x, aggp, degp, ws, wn, b, residual):
    n, din = x.shape
    dout = ws.shape[1]
    r = 512
    grid = (pl.cdiv(n, r),)

    def body(x_ref, a_ref, d_ref, ws_ref, wn_ref, b_ref, o_ref):
        agg = a_ref[0] + a_ref[1]
        deg = jnp.sum(d_ref[...], axis=0)[:, None]
        mean = agg / jnp.maximum(deg, 1.0)
        h = jnp.dot(x_ref[...], ws_ref[...], preferred_element_type=jnp.float32)
        h = h + jnp.dot(mean, wn_ref[...], preferred_element_type=jnp.float32)
        h = jnp.maximum(h + b_ref[...], 0.0)
        if residual:
            h = h + x_ref[...]
        o_ref[...] = h

    return pl.pallas_call(
        body,
        grid=grid,
        in_specs=[
            pl.BlockSpec((r, din), lambda i: (i, 0)),
            pl.BlockSpec((NC, r, din), lambda i: (0, i, 0)),
            pl.BlockSpec((NW, r), lambda i: (0, i)),
            pl.BlockSpec((din, dout), lambda i: (0, 0)),
            pl.BlockSpec((din, dout), lambda i: (0, 0)),
            pl.BlockSpec((1, dout), lambda i: (0, 0)),
        ],
        out_specs=pl.BlockSpec((r, dout), lambda i: (i, 0)),
        out_shape=jax.ShapeDtypeStruct((n, dout), jnp.float32),
    )(x, aggp, degp, ws, wn, b.reshape(1, dout))


def _attn_fuse(mob, ed, mask, p):
    n_mob, o_mob = mob.shape
    n_ed, o_ed = ed.shape
    r = 1024
    grid = (pl.cdiv(n_mob, r),)

    def body(mob_ref, ed_ref, mask_ref, wq_ref, wk_ref, alpha_ref,
             wf1_ref, bf1_ref, wf2_ref, bf2_ref, wg_ref, bg_ref,
             lng_ref, lnb_ref, o_ref):
        m = mob_ref[...]
        ed_f = ed_ref[...]
        q = jnp.dot(m, wq_ref[...], preferred_element_type=jnp.float32)
        kk = jnp.dot(ed_f, wk_ref[...], preferred_element_type=jnp.float32)
        scores = lax.dot_general(
            q, kk, (((1,), (1,)), ((), ())),
            preferred_element_type=jnp.float32)
        smax = jnp.max(scores, axis=1, keepdims=True)
        e = jnp.exp(scores - smax)
        learned = e / jnp.sum(e, axis=1, keepdims=True)
        msk = mask_ref[...]
        mask_norm = msk / jnp.maximum(jnp.sum(msk, axis=1, keepdims=True), 1e-8)
        g = 1.0 / (1.0 + jnp.exp(-alpha_ref[0, 0]))
        attn = g * mask_norm + (1.0 - g) * learned
        sig = jnp.dot(attn, ed_f, preferred_element_type=jnp.float32)

        wf1 = wf1_ref[...]
        h1 = (jnp.dot(m, wf1[:o_mob, :], preferred_element_type=jnp.float32)
              + jnp.dot(sig, wf1[o_mob:o_mob + o_ed, :],
                        preferred_element_type=jnp.float32) + bf1_ref[...])
        h1 = jnp.maximum(h1, 0.0)
        fused = jnp.dot(h1, wf2_ref[...],
                        preferred_element_type=jnp.float32) + bf2_ref[...]
        wg = wg_ref[...]
        gz = (jnp.dot(m, wg[:o_mob, :], preferred_element_type=jnp.float32)
              + jnp.dot(sig, wg[o_mob:o_mob + o_ed, :],
                        preferred_element_type=jnp.float32) + bg_ref[...])
        gate = 1.0 / (1.0 + jnp.exp(-gz))
        comb = gate * fused + (1.0 - gate) * m
        mu = jnp.mean(comb, axis=-1, keepdims=True)
        var = jnp.mean((comb - mu) ** 2, axis=-1, keepdims=True)
        comb = (comb - mu) / jnp.sqrt(var + 1e-5) * lng_ref[...] + lnb_ref[...]
        nrm = jnp.sqrt(jnp.sum(comb * comb, axis=-1, keepdims=True))
        o_ref[...] = comb / jnp.maximum(nrm, 1e-12)

    o_ed_dim = p['Wk'].shape[1]
    return pl.pallas_call(
        body,
        grid=grid,
        in_specs=[
            pl.BlockSpec((r, o_mob), lambda i: (i, 0)),
            pl.BlockSpec((n_ed, o_ed), lambda i: (0, 0)),
            pl.BlockSpec((r, n_ed), lambda i: (i, 0)),
            pl.BlockSpec(p['Wq'].shape, lambda i: (0, 0)),
            pl.BlockSpec(p['Wk'].shape, lambda i: (0, 0)),
            pl.BlockSpec((1, 1), lambda i: (0, 0)),
            pl.BlockSpec(p['Wf1'].shape, lambda i: (0, 0)),
            pl.BlockSpec((1, o_mob), lambda i: (0, 0)),
            pl.BlockSpec(p['Wf2'].shape, lambda i: (0, 0)),
            pl.BlockSpec((1, o_mob), lambda i: (0, 0)),
            pl.BlockSpec(p['Wg'].shape, lambda i: (0, 0)),
            pl.BlockSpec((1, o_mob), lambda i: (0, 0)),
            pl.BlockSpec((1, o_mob), lambda i: (0, 0)),
            pl.BlockSpec((1, o_mob), lambda i: (0, 0)),
        ],
        out_specs=pl.BlockSpec((r, o_mob), lambda i: (i, 0)),
        out_shape=jax.ShapeDtypeStruct((n_mob, o_mob), jnp.float32),
    )(mob, ed, mask,
      p['Wq'], p['Wk'], p['alpha'].reshape(1, 1),
      p['Wf1'], p['bf1'].reshape(1, o_mob), p['Wf2'], p['bf2'].reshape(1, o_mob),
      p['Wg'], p['bg'].reshape(1, o_mob),
      p['ln_g'].reshape(1, o_mob), p['ln_b'].reshape(1, o_mob))


def _sage2(x, ei, layers):
    n, din = x.shape
    e = ei.shape[1]
    h = x
    degp = None
    for ws, wn, b, residual in layers:
        d = h.shape[1]
        dpad = -(-d // 128) * 128
        hp = h if dpad == d else jnp.pad(h, ((0, 0), (0, dpad - d)))
        zer = jnp.zeros((n, dpad), jnp.float32)
        if degp is None:
            aggp, degp = _seg_sum_sc(n, dpad, e, True)(hp, ei, zer)
            degp = degp.reshape(NW, n)
        else:
            aggp = _seg_sum_sc(n, dpad, e, False)(hp, ei, zer)
        if dpad != d:
            aggp = aggp[:, :, :d]
        h = _sage_dense(h, aggp, degp, ws, wn, b, residual)
    return h


def kernel(mobility_x, mobility_edge_index, edar_x, edar_edge_index,
           edar_muni_mask, params):
    p = params

    mob = _sage2(mobility_x, mobility_edge_index, [
        (p['mob_Ws1'], p['mob_Wn1'], p['mob_b1'], True),
        (p['mob_Ws2'], p['mob_Wn2'], p['mob_b2'], False),
    ])
    ed = _sage2(edar_x, edar_edge_index, [
        (p['ed_Ws1'], p['ed_Wn1'], p['ed_b1'], False),
        (p['ed_Ws2'], p['ed_Wn2'], p['ed_b2'], False),
    ])
    return _attn_fuse(mob, ed, edar_muni_mask, p)


# speedup vs baseline: 7.8551x; 1.0045x over previous
import functools

import jax
import jax.numpy as jnp
from jax import lax
from jax.experimental import pallas as pl
from jax.experimental.pallas import tpu as pltpu
from jax.experimental.pallas import tpu_sc as plsc

NC = 2
NS = 16
L = 16
NW = NC * NS
CH = 128
NB = 2
NE = 4
P = 2


@functools.lru_cache(maxsize=None)
def _seg_sum_sc(n_nodes, dim, n_edges, with_deg=True):
    assert n_nodes % 8 == 0 and n_edges % CH == 0 and dim % L == 0
    full = (-(-n_nodes // NS) + 7) // 8 * 8
    tail = n_nodes - full * (NS - 1)
    assert 0 < tail <= full
    n_chunks = n_edges // CH
    base_chunks = n_chunks // NW
    rem = n_chunks % NW

    mesh = plsc.VectorSubcoreMesh(core_axis_name="c", subcore_axis_name="s",
                                  num_cores=NC, num_subcores=NS)
    agg_t = jax.ShapeDtypeStruct((NC, n_nodes, dim), jnp.float32)
    out_type = ([agg_t, jax.ShapeDtypeStruct((NW * n_nodes,), jnp.float32)]
                if with_deg else agg_t)
    scratch = [
        pltpu.VMEM((NE, 2, CH), jnp.int32),
        pltpu.VMEM((NB, CH, dim), jnp.float32),
        pltpu.VMEM_SHARED((n_nodes, dim), jnp.float32),
        pltpu.SemaphoreType.DMA((NE,)),
        pltpu.SemaphoreType.DMA((NB,)),
        pltpu.SemaphoreType.DMA((NB,)),
    ]
    if with_deg:
        scratch.append(pltpu.VMEM((n_nodes,), jnp.float32))

    def impl(x_hbm, ei_hbm, zeros_hbm, agg_out,
             eib, rows, agg_sh, sem_e, sem_g, sem_sc,
             deg_out, degv, zeros1_hbm):
        c = lax.axis_index("c")
        s = lax.axis_index("s")
        wid = s * NC + c

        my_n = base_chunks + jnp.where(wid < rem, 1, 0)
        my_first = wid * base_chunks + jnp.minimum(wid, rem)

        if with_deg:
            pltpu.sync_copy(zeros1_hbm, degv)

        r0 = s * full

        @pl.when(s < NS - 1)
        def _():
            pltpu.sync_copy(zeros_hbm.at[pl.ds(r0, full)],
                            agg_sh.at[pl.ds(r0, full)])

        @pl.when(s == NS - 1)
        def _():
            pltpu.sync_copy(zeros_hbm.at[pl.ds(r0, tail)],
                            agg_sh.at[pl.ds(r0, tail)])

        plsc.subcore_barrier()

        ones16 = jnp.ones((L,), jnp.float32)

        def fetch_idx(j):
            pltpu.async_copy(
                ei_hbm.at[pl.ds(0, 2), pl.ds((my_first + j) * CH, CH)],
                eib.at[lax.rem(j, NE)], sem_e.at[lax.rem(j, NE)])

        def wait_idx(j):
            pltpu.make_async_copy(
                ei_hbm.at[pl.ds(0, 2), pl.ds((my_first + j) * CH, CH)],
                eib.at[lax.rem(j, NE)], sem_e.at[lax.rem(j, NE)]).wait()

        def fire_gather(j):
            pltpu.async_copy(x_hbm.at[eib.at[lax.rem(j, NE), 0]],
                             rows.at[lax.rem(j, NB)],
                             sem_g.at[lax.rem(j, NB)])

        def wait_gather(j):
            pltpu.make_async_copy(x_hbm.at[eib.at[lax.rem(j, NE), 0]],
                                  rows.at[lax.rem(j, NB)],
                                  sem_g.at[lax.rem(j, NB)]).wait()

        def wait_scatter_slot(slot):
            pltpu.make_async_copy(rows.at[slot], agg_sh.at[eib.at[0, 1]],
                                  sem_sc.at[slot]).wait()

        for kk in range(P):
            @pl.when(kk < my_n)
            def _(kk=kk):
                fetch_idx(kk)

        for kk in range(P - 1):
            @pl.when(kk < my_n)
            def _(kk=kk):
                wait_idx(kk)
                fire_gather(kk)

        def body(i, _):
            @pl.when(i + P < my_n)
            def _():
                fetch_idx(i + P)

            wait_gather(i)
            e0 = lax.rem(i, NE)
            pltpu.async_copy(rows.at[lax.rem(i, NB)],
                             agg_sh.at[eib.at[e0, 1]],
                             sem_sc.at[lax.rem(i, NB)], add=True)
            if with_deg:
                for j in range(CH // L):
                    idx = eib[e0, 1, pl.ds(j * L, L)]
                    plsc.addupdate_scatter(degv, [idx], ones16)

            @pl.when(i + P - 1 < my_n)
            def _():
                wait_idx(i + P - 1)

                @pl.when(i + P - 1 >= NB)
                def _():
                    wait_scatter_slot(lax.rem(i + P - 1, NB))

                fire_gather(i + P - 1)
            return 0

        lax.fori_loop(0, my_n, body, 0)

        for kk in range(1, NB + 1):
            @pl.when(my_n >= kk)
            def _(kk=kk):
                wait_scatter_slot(lax.rem(my_n - kk + 2 * NB, NB))

        if with_deg:
            pltpu.sync_copy(degv, deg_out.at[pl.ds(wid * n_nodes, n_nodes)])
        plsc.subcore_barrier()

        @pl.when(s < NS - 1)
        def _():
            pltpu.sync_copy(agg_sh.at[pl.ds(r0, full)],
                            agg_out.at[c, pl.ds(r0, full)])

        @pl.when(s == NS - 1)
        def _():
            pltpu.sync_copy(agg_sh.at[pl.ds(r0, tail)],
                            agg_out.at[c, pl.ds(r0, tail)])

    kw = dict(mesh=mesh, out_type=out_type, scratch_types=scratch,
              compiler_params=pltpu.CompilerParams(needs_layout_passes=False))
    if with_deg:
        @functools.partial(pl.kernel, **kw)
        def k(x_hbm, ei_hbm, zeros_hbm, zeros1_hbm, agg_out, deg_out,
              eib, rows, agg_sh, sem_e, sem_g, sem_sc, degv):
            impl(x_hbm, ei_hbm, zeros_hbm, agg_out,
                 eib, rows, agg_sh, sem_e, sem_g, sem_sc,
                 deg_out, degv, zeros1_hbm)
    else:
        @functools.partial(pl.kernel, **kw)
        def k(x_hbm, ei_hbm, zeros_hbm, agg_out,
              eib, rows, agg_sh, sem_e, sem_g, sem_sc):
            impl(x_hbm, ei_hbm, zeros_hbm, agg_out,
                 eib, rows, agg_sh, sem_e, sem_g, sem_sc, None, None, None)

    return k


def _sage_dense(x, aggp, degp, ws, wn, b, residual):
    n, din = x.shape
    dout = ws.shape[1]
    r = 1024
    grid = (pl.cdiv(n, r),)

    def body(x_ref, a_ref, d_ref, ws_ref, wn_ref, b_ref, o_ref):
        agg = a_ref[0] + a_ref[1]
        deg = jnp.sum(d_ref[...], axis=0)[:, None]
        mean = agg / jnp.maximum(deg, 1.0)
        h = jnp.dot(x_ref[...], ws_ref[...], preferred_element_type=jnp.float32)
        h = h + jnp.dot(mean, wn_ref[...], preferred_element_type=jnp.float32)
        h = jnp.maximum(h + b_ref[...], 0.0)
        if residual:
            h = h + x_ref[...]
        o_ref[...] = h

    return pl.pallas_call(
        body,
        grid=grid,
        in_specs=[
            pl.BlockSpec((r, din), lambda i: (i, 0)),
            pl.BlockSpec((NC, r, din), lambda i: (0, i, 0)),
            pl.BlockSpec((NW, r), lambda i: (0, i)),
            pl.BlockSpec((din, dout), lambda i: (0, 0)),
            pl.BlockSpec((din, dout), lambda i: (0, 0)),
            pl.BlockSpec((1, dout), lambda i: (0, 0)),
        ],
        out_specs=pl.BlockSpec((r, dout), lambda i: (i, 0)),
        out_shape=jax.ShapeDtypeStruct((n, dout), jnp.float32),
    )(x, aggp, degp, ws, wn, b.reshape(1, dout))


def _attn_fuse(mob, ed, mask, p):
    n_mob, o_mob = mob.shape
    n_ed, o_ed = ed.shape
    r = 1024
    grid = (pl.cdiv(n_mob, r),)

    def body(mob_ref, ed_ref, mask_ref, wq_ref, wk_ref, alpha_ref,
             wf1_ref, bf1_ref, wf2_ref, bf2_ref, wg_ref, bg_ref,
             lng_ref, lnb_ref, o_ref):
        m = mob_ref[...]
        ed_f = ed_ref[...]
        q = jnp.dot(m, wq_ref[...], preferred_element_type=jnp.float32)
        kk = jnp.dot(ed_f, wk_ref[...], preferred_element_type=jnp.float32)
        scores = lax.dot_general(
            q, kk, (((1,), (1,)), ((), ())),
            preferred_element_type=jnp.float32)
        smax = jnp.max(scores, axis=1, keepdims=True)
        e = jnp.exp(scores - smax)
        learned = e / jnp.sum(e, axis=1, keepdims=True)
        msk = mask_ref[...]
        mask_norm = msk / jnp.maximum(jnp.sum(msk, axis=1, keepdims=True), 1e-8)
        g = 1.0 / (1.0 + jnp.exp(-alpha_ref[0, 0]))
        attn = g * mask_norm + (1.0 - g) * learned
        sig = jnp.dot(attn, ed_f, preferred_element_type=jnp.float32)

        wf1 = wf1_ref[...]
        h1 = (jnp.dot(m, wf1[:o_mob, :], preferred_element_type=jnp.float32)
              + jnp.dot(sig, wf1[o_mob:o_mob + o_ed, :],
                        preferred_element_type=jnp.float32) + bf1_ref[...])
        h1 = jnp.maximum(h1, 0.0)
        fused = jnp.dot(h1, wf2_ref[...],
                        preferred_element_type=jnp.float32) + bf2_ref[...]
        wg = wg_ref[...]
        gz = (jnp.dot(m, wg[:o_mob, :], preferred_element_type=jnp.float32)
              + jnp.dot(sig, wg[o_mob:o_mob + o_ed, :],
                        preferred_element_type=jnp.float32) + bg_ref[...])
        gate = 1.0 / (1.0 + jnp.exp(-gz))
        comb = gate * fused + (1.0 - gate) * m
        mu = jnp.mean(comb, axis=-1, keepdims=True)
        var = jnp.mean((comb - mu) ** 2, axis=-1, keepdims=True)
        comb = (comb - mu) / jnp.sqrt(var + 1e-5) * lng_ref[...] + lnb_ref[...]
        nrm = jnp.sqrt(jnp.sum(comb * comb, axis=-1, keepdims=True))
        o_ref[...] = comb / jnp.maximum(nrm, 1e-12)

    o_ed_dim = p['Wk'].shape[1]
    return pl.pallas_call(
        body,
        grid=grid,
        in_specs=[
            pl.BlockSpec((r, o_mob), lambda i: (i, 0)),
            pl.BlockSpec((n_ed, o_ed), lambda i: (0, 0)),
            pl.BlockSpec((r, n_ed), lambda i: (i, 0)),
            pl.BlockSpec(p['Wq'].shape, lambda i: (0, 0)),
            pl.BlockSpec(p['Wk'].shape, lambda i: (0, 0)),
            pl.BlockSpec((1, 1), lambda i: (0, 0)),
            pl.BlockSpec(p['Wf1'].shape, lambda i: (0, 0)),
            pl.BlockSpec((1, o_mob), lambda i: (0, 0)),
            pl.BlockSpec(p['Wf2'].shape, lambda i: (0, 0)),
            pl.BlockSpec((1, o_mob), lambda i: (0, 0)),
            pl.BlockSpec(p['Wg'].shape, lambda i: (0, 0)),
            pl.BlockSpec((1, o_mob), lambda i: (0, 0)),
            pl.BlockSpec((1, o_mob), lambda i: (0, 0)),
            pl.BlockSpec((1, o_mob), lambda i: (0, 0)),
        ],
        out_specs=pl.BlockSpec((r, o_mob), lambda i: (i, 0)),
        out_shape=jax.ShapeDtypeStruct((n_mob, o_mob), jnp.float32),
    )(mob, ed, mask,
      p['Wq'], p['Wk'], p['alpha'].reshape(1, 1),
      p['Wf1'], p['bf1'].reshape(1, o_mob), p['Wf2'], p['bf2'].reshape(1, o_mob),
      p['Wg'], p['bg'].reshape(1, o_mob),
      p['ln_g'].reshape(1, o_mob), p['ln_b'].reshape(1, o_mob))


def _sage2(x, ei, layers):
    n, din = x.shape
    e = ei.shape[1]
    h = x
    degp = None
    for ws, wn, b, residual in layers:
        d = h.shape[1]
        dpad = -(-d // 128) * 128
        hp = h if dpad == d else jnp.pad(h, ((0, 0), (0, dpad - d)))
        zer = jnp.zeros((n, dpad), jnp.float32)
        if degp is None:
            zer1 = jnp.zeros((n,), jnp.float32)
            aggp, degp = _seg_sum_sc(n, dpad, e, True)(hp, ei, zer, zer1)
            degp = degp.reshape(NW, n)
        else:
            aggp = _seg_sum_sc(n, dpad, e, False)(hp, ei, zer)
        if dpad != d:
            aggp = aggp[:, :, :d]
        h = _sage_dense(h, aggp, degp, ws, wn, b, residual)
    return h


def kernel(mobility_x, mobility_edge_index, edar_x, edar_edge_index,
           edar_muni_mask, params):
    p = params

    mob = _sage2(mobility_x, mobility_edge_index, [
        (p['mob_Ws1'], p['mob_Wn1'], p['mob_b1'], True),
        (p['mob_Ws2'], p['mob_Wn2'], p['mob_b2'], False),
    ])
    ed = _sage2(edar_x, edar_edge_index, [
        (p['ed_Ws1'], p['ed_Wn1'], p['ed_b1'], False),
        (p['ed_Ws2'], p['ed_Wn2'], p['ed_b2'], False),
    ])
    return _attn_fuse(mob, ed, edar_muni_mask, p)
